# baseline (speedup 1.0000x reference)
import functools

import jax
import jax.numpy as jnp
from jax import lax
from jax.experimental import pallas as pl
from jax.experimental.pallas import tpu as pltpu
from jax.experimental.pallas import tpu_sc as plsc

N_NODES = 10000
C = 128
E = 320000
EPS = 1e-5

NC, NS, L = 2, 16, 16
NW = NC * NS
EPT = E // NW
CHUNK = 80
NCHUNK = EPT // CHUNK
NBUF = 3
N_PAD = 10112
STRIPE = N_PAD // NS


_sc_mesh = plsc.VectorSubcoreMesh(
    core_axis_name="c", subcore_axis_name="s", num_cores=NC, num_subcores=NS
)


def _sc_body(xt, g_hbm, r_hbm, out_s, out_c, acc, cnt,
             rows0, rows1, rows2, gb0, gb1, gb2, rb0, rb1, rb2,
             gs0, gs1, gs2, ss0, ss1, ss2,
             ig0, ig1, ig2, ir0, ir1, ir2):
    c = lax.axis_index("c")
    s = lax.axis_index("s")
    wid = c * NS + s
    ROWS = (rows0, rows1, rows2)
    GB = (gb0, gb1, gb2)
    RB = (rb0, rb1, rb2)
    GS = (gs0, gs1, gs2)
    SS = (ss0, ss1, ss2)
    IG = (ig0, ig1, ig2)
    IR = (ir0, ir1, ir2)

    ebase = wid * EPT

    def ig_start(a, b):
        pltpu.async_copy(g_hbm.at[pl.ds(ebase + a * CHUNK, CHUNK)], GB[b], IG[b])

    def ig_wait(b):
        pltpu.make_async_copy(g_hbm.at[pl.ds(0, CHUNK)], GB[b], IG[b]).wait()

    def ir_start(a, b):
        pltpu.async_copy(r_hbm.at[pl.ds(ebase + a * CHUNK, CHUNK)], RB[b], IR[b])

    def ir_wait(b):
        pltpu.make_async_copy(r_hbm.at[pl.ds(0, CHUNK)], RB[b], IR[b]).wait()

    def g_start(b):
        pltpu.async_copy(xt.at[GB[b]], ROWS[b], GS[b])

    def g_wait(b):
        pltpu.make_async_copy(xt.at[GB[b]], ROWS[b], GS[b]).wait()

    def s_wait(b):
        pltpu.make_async_copy(ROWS[b], acc.at[RB[b]], SS[b]).wait()

    ig_start(0, 0)
    ig_start(1, 1)
    ir_start(0, 0)
    ir_start(1, 1)
    ig_wait(0)
    g_start(0)
    ig_wait(1)
    g_start(1)

    zv = jnp.zeros((L,), jnp.float32)

    def zrow(i, carry):
        rows2[i // (C // L), pl.ds((i % (C // L)) * L, L)] = zv
        return carry

    lax.fori_loop(0, CHUNK * (C // L), zrow, 0)
    for k in range(STRIPE // CHUNK):
        pltpu.sync_copy(rows2, acc.at[pl.ds(s * STRIPE + k * CHUNK, CHUNK)])
    rem = STRIPE % CHUNK
    if rem:
        pltpu.sync_copy(
            rows2.at[pl.ds(0, rem)],
            acc.at[pl.ds(s * STRIPE + (STRIPE // CHUNK) * CHUNK, rem)],
        )

    def zcnt(i, carry):
        cnt[pl.ds(i * L, L)] = zv
        return carry

    lax.fori_loop(0, N_PAD // L, zcnt, 0)
    plsc.subcore_barrier()

    def counts(b):
        for k in range(CHUNK // L):
            v = RB[b][pl.ds(k * L, L)]
            r, last = plsc.scan_count(v)
            plsc.addupdate_scatter(cnt, [v], r.astype(jnp.float32), mask=last)

    def blocks(j, carry):
        for b in range(NBUF):
            a = NBUF * j + b

            @pl.when(a < NCHUNK)
            def _():
                b2 = (b + 2) % NBUF

                @pl.when(a + 2 < NCHUNK)
                def _():
                    ig_start(a + 2, b2)

                g_wait(b)
                ir_wait(b)
                pltpu.async_copy(ROWS[b], acc.at[RB[b]], SS[b], add=True)

                @pl.when(a + 2 < NCHUNK)
                def _():
                    @pl.when(a >= 1)
                    def _():
                        s_wait(b2)

                    ir_start(a + 2, b2)
                    ig_wait(b2)
                    g_start(b2)

                counts(b)

        return carry

    lax.fori_loop(0, (NCHUNK + NBUF - 1) // NBUF, blocks, 0)
    s_wait(0)
    s_wait(1)
    s_wait(2)

    plsc.subcore_barrier()
    pltpu.sync_copy(
        acc.at[pl.ds(s * STRIPE, STRIPE)],
        out_s.at[c, pl.ds(s * STRIPE, STRIPE)],
    )
    pltpu.sync_copy(cnt, out_c.at[c, s])


_sc_aggregate = functools.partial(
    pl.kernel,
    out_type=(
        jax.ShapeDtypeStruct((NC, N_PAD, C), jnp.float32),
        jax.ShapeDtypeStruct((NC, NS, N_PAD), jnp.float32),
    ),
    mesh=_sc_mesh,
    compiler_params=pltpu.CompilerParams(needs_layout_passes=False),
    scratch_types=[
        pltpu.VMEM_SHARED((N_PAD, C), jnp.float32),
        pltpu.VMEM((N_PAD,), jnp.float32),
        pltpu.VMEM((CHUNK, C), jnp.float32),
        pltpu.VMEM((CHUNK, C), jnp.float32),
        pltpu.VMEM((CHUNK, C), jnp.float32),
        pltpu.VMEM((CHUNK,), jnp.int32),
        pltpu.VMEM((CHUNK,), jnp.int32),
        pltpu.VMEM((CHUNK,), jnp.int32),
        pltpu.VMEM((CHUNK,), jnp.int32),
        pltpu.VMEM((CHUNK,), jnp.int32),
        pltpu.VMEM((CHUNK,), jnp.int32),
        pltpu.SemaphoreType.DMA,
        pltpu.SemaphoreType.DMA,
        pltpu.SemaphoreType.DMA,
        pltpu.SemaphoreType.DMA,
        pltpu.SemaphoreType.DMA,
        pltpu.SemaphoreType.DMA,
        pltpu.SemaphoreType.DMA,
        pltpu.SemaphoreType.DMA,
        pltpu.SemaphoreType.DMA,
        pltpu.SemaphoreType.DMA,
        pltpu.SemaphoreType.DMA,
        pltpu.SemaphoreType.DMA,
    ],
)(_sc_body)


def _pre_body(x_ref, wv_ref, xt_ref, zv_ref):
    x = x_ref[...]
    xt_ref[...] = x.T
    zv_ref[...] = lax.dot_general(wv_ref[...], x, (((1,), (0,)), ((), ())),
                                  preferred_element_type=jnp.float32)


_tc_pre = pl.pallas_call(
    _pre_body,
    out_shape=(
        jax.ShapeDtypeStruct((N_NODES, C), jnp.float32),
        jax.ShapeDtypeStruct((C, N_NODES), jnp.float32),
    ),
)


def _tc_body(zv_ref, p_ref, c_ref, wn_ref, g_ref, b_ref, a_ref, o_ref):
    sums = p_ref[0, :N_NODES, :] + p_ref[1, :N_NODES, :]
    counts = jnp.sum(c_ref[...], axis=(0, 1))[:N_NODES]
    inv = 1.0 / jnp.maximum(counts, 1.0)
    zn = lax.dot_general(wn_ref[...], sums, (((1,), (1,)), ((), ())),
                         preferred_element_type=jnp.float32)
    z = zv_ref[...] + zn * inv[None, :]
    mu = jnp.mean(z, axis=1, keepdims=True)
    var = jnp.mean(z * z, axis=1, keepdims=True) - mu * mu
    scale = g_ref[...] * lax.rsqrt(var + EPS)
    out = (z - mu) * scale + b_ref[...]
    alpha = a_ref[0, 0]
    o_ref[...] = jnp.where(out >= 0, out, alpha * out)


_tc_finish = pl.pallas_call(
    _tc_body,
    out_shape=jax.ShapeDtypeStruct((C, N_NODES), jnp.float32),
    in_specs=[
        pl.BlockSpec(memory_space=pltpu.VMEM),
        pl.BlockSpec(memory_space=pltpu.VMEM),
        pl.BlockSpec(memory_space=pltpu.VMEM),
        pl.BlockSpec(memory_space=pltpu.VMEM),
        pl.BlockSpec(memory_space=pltpu.VMEM),
        pl.BlockSpec(memory_space=pltpu.VMEM),
        pl.BlockSpec(memory_space=pltpu.SMEM),
    ],
)


def kernel(in_features, W_v, W_n, gamma, beta, alpha, reduce_index, gather_index):
    x = in_features[0]
    xt, zv = _tc_pre(x, W_v)
    psums, pcnts = _sc_aggregate(xt, gather_index, reduce_index)
    out = _tc_finish(
        zv, psums, pcnts, W_n,
        gamma.reshape(C, 1), beta.reshape(C, 1),
        jnp.reshape(alpha, (1, 1)),
    )
    return out[None]


# speedup vs baseline: 1.0710x; 1.0710x over previous
import functools

import jax
import jax.numpy as jnp
from jax import lax
from jax.experimental import pallas as pl
from jax.experimental.pallas import tpu as pltpu
from jax.experimental.pallas import tpu_sc as plsc

N_NODES = 10000
C = 128
E = 320000
EPS = 1e-5

NC, NS, L = 2, 16, 16
NW = NC * NS
EPT = E // NW
CHUNK = 80
NCHUNK = EPT // CHUNK
NBUF = 3
N_PAD = 10112
STRIPE = N_PAD // NS


_sc_mesh = plsc.VectorSubcoreMesh(
    core_axis_name="c", subcore_axis_name="s", num_cores=NC, num_subcores=NS
)


def _sc_body(xt, g_hbm, r_hbm, out_s, out_c, acc, cnt,
             rows0, rows1, rows2, gb0, gb1, gb2, rb0, rb1, rb2,
             gs0, gs1, gs2, ss0, ss1, ss2,
             ig0, ig1, ig2, ir0, ir1, ir2):
    c = lax.axis_index("c")
    s = lax.axis_index("s")
    wid = c * NS + s
    ROWS = (rows0, rows1, rows2)
    GB = (gb0, gb1, gb2)
    RB = (rb0, rb1, rb2)
    GS = (gs0, gs1, gs2)
    SS = (ss0, ss1, ss2)
    IG = (ig0, ig1, ig2)
    IR = (ir0, ir1, ir2)

    ebase = wid * EPT

    def ig_start(a, b):
        pltpu.async_copy(g_hbm.at[pl.ds(ebase + a * CHUNK, CHUNK)], GB[b], IG[b])

    def ig_wait(b):
        pltpu.make_async_copy(g_hbm.at[pl.ds(0, CHUNK)], GB[b], IG[b]).wait()

    def ir_start(a, b):
        pltpu.async_copy(r_hbm.at[pl.ds(ebase + a * CHUNK, CHUNK)], RB[b], IR[b])

    def ir_wait(b):
        pltpu.make_async_copy(r_hbm.at[pl.ds(0, CHUNK)], RB[b], IR[b]).wait()

    def g_start(b):
        pltpu.async_copy(xt.at[GB[b]], ROWS[b], GS[b])

    def g_wait(b):
        pltpu.make_async_copy(xt.at[GB[b]], ROWS[b], GS[b]).wait()

    def s_wait(b):
        pltpu.make_async_copy(ROWS[b], acc.at[RB[b]], SS[b]).wait()

    ig_start(0, 0)
    ig_start(1, 1)
    ir_start(0, 0)
    ir_start(1, 1)
    ig_wait(0)
    g_start(0)
    ig_wait(1)
    g_start(1)

    zv = jnp.zeros((L,), jnp.float32)

    def zrow(i, carry):
        rows2[i // (C // L), pl.ds((i % (C // L)) * L, L)] = zv
        return carry

    lax.fori_loop(0, CHUNK * (C // L), zrow, 0)
    for k in range(STRIPE // CHUNK):
        pltpu.sync_copy(rows2, acc.at[pl.ds(s * STRIPE + k * CHUNK, CHUNK)])
    rem = STRIPE % CHUNK
    if rem:
        pltpu.sync_copy(
            rows2.at[pl.ds(0, rem)],
            acc.at[pl.ds(s * STRIPE + (STRIPE // CHUNK) * CHUNK, rem)],
        )

    def zcnt(i, carry):
        cnt[pl.ds(i * L, L)] = zv
        return carry

    lax.fori_loop(0, N_PAD // L, zcnt, 0)
    plsc.subcore_barrier()

    def counts(b):
        for k in range(CHUNK // L):
            v = RB[b][pl.ds(k * L, L)]
            r, last = plsc.scan_count(v)
            plsc.addupdate_scatter(cnt, [v], r.astype(jnp.float32), mask=last)

    def blocks(j, carry):
        for b in range(NBUF):
            a = NBUF * j + b

            @pl.when(a < NCHUNK)
            def _():
                b2 = (b + 2) % NBUF

                @pl.when(a + 2 < NCHUNK)
                def _():
                    ig_start(a + 2, b2)

                g_wait(b)
                ir_wait(b)
                pltpu.async_copy(ROWS[b], acc.at[RB[b]], SS[b], add=True)

                @pl.when(a + 2 < NCHUNK)
                def _():
                    @pl.when(a >= 1)
                    def _():
                        s_wait(b2)

                    ir_start(a + 2, b2)
                    ig_wait(b2)
                    g_start(b2)

                counts(b)

        return carry

    lax.fori_loop(0, (NCHUNK + NBUF - 1) // NBUF, blocks, 0)
    s_wait(0)
    s_wait(1)
    s_wait(2)

    plsc.subcore_barrier()
    pltpu.sync_copy(
        acc.at[pl.ds(s * STRIPE, STRIPE)],
        out_s.at[c, pl.ds(s * STRIPE, STRIPE)],
    )
    pltpu.sync_copy(cnt, out_c.at[c, s])


_sc_aggregate = functools.partial(
    pl.kernel,
    out_type=(
        jax.ShapeDtypeStruct((NC, N_PAD, C), jnp.float32),
        jax.ShapeDtypeStruct((NC, NS, N_PAD), jnp.float32),
    ),
    mesh=_sc_mesh,
    compiler_params=pltpu.CompilerParams(needs_layout_passes=False),
    scratch_types=[
        pltpu.VMEM_SHARED((N_PAD, C), jnp.float32),
        pltpu.VMEM((N_PAD,), jnp.float32),
        pltpu.VMEM((CHUNK, C), jnp.float32),
        pltpu.VMEM((CHUNK, C), jnp.float32),
        pltpu.VMEM((CHUNK, C), jnp.float32),
        pltpu.VMEM((CHUNK,), jnp.int32),
        pltpu.VMEM((CHUNK,), jnp.int32),
        pltpu.VMEM((CHUNK,), jnp.int32),
        pltpu.VMEM((CHUNK,), jnp.int32),
        pltpu.VMEM((CHUNK,), jnp.int32),
        pltpu.VMEM((CHUNK,), jnp.int32),
        pltpu.SemaphoreType.DMA,
        pltpu.SemaphoreType.DMA,
        pltpu.SemaphoreType.DMA,
        pltpu.SemaphoreType.DMA,
        pltpu.SemaphoreType.DMA,
        pltpu.SemaphoreType.DMA,
        pltpu.SemaphoreType.DMA,
        pltpu.SemaphoreType.DMA,
        pltpu.SemaphoreType.DMA,
        pltpu.SemaphoreType.DMA,
        pltpu.SemaphoreType.DMA,
        pltpu.SemaphoreType.DMA,
    ],
)(_sc_body)


def _tc_body(x_ref, p_ref, c_ref, wv_ref, wn_ref, g_ref, b_ref, a_ref, o_ref):
    x = x_ref[...]
    sums = p_ref[0, :N_NODES, :] + p_ref[1, :N_NODES, :]
    counts = jnp.sum(c_ref[...], axis=(0, 1))[:N_NODES]
    inv = 1.0 / jnp.maximum(counts, 1.0)
    zn = lax.dot_general(wn_ref[...], sums, (((1,), (1,)), ((), ())),
                         preferred_element_type=jnp.float32)
    zv = lax.dot_general(wv_ref[...], x, (((1,), (0,)), ((), ())),
                         preferred_element_type=jnp.float32)
    z = zv + zn * inv[None, :]
    mu = jnp.mean(z, axis=1, keepdims=True)
    var = jnp.mean(z * z, axis=1, keepdims=True) - mu * mu
    scale = g_ref[...] * lax.rsqrt(var + EPS)
    out = (z - mu) * scale + b_ref[...]
    alpha = a_ref[0, 0]
    o_ref[...] = jnp.where(out >= 0, out, alpha * out)


_tc_finish = pl.pallas_call(
    _tc_body,
    out_shape=jax.ShapeDtypeStruct((C, N_NODES), jnp.float32),
    in_specs=[
        pl.BlockSpec(memory_space=pltpu.VMEM),
        pl.BlockSpec(memory_space=pltpu.VMEM),
        pl.BlockSpec(memory_space=pltpu.VMEM),
        pl.BlockSpec(memory_space=pltpu.VMEM),
        pl.BlockSpec(memory_space=pltpu.VMEM),
        pl.BlockSpec(memory_space=pltpu.VMEM),
        pl.BlockSpec(memory_space=pltpu.VMEM),
        pl.BlockSpec(memory_space=pltpu.SMEM),
    ],
)


def kernel(in_features, W_v, W_n, gamma, beta, alpha, reduce_index, gather_index):
    x = in_features[0]
    xt = x.T
    psums, pcnts = _sc_aggregate(xt, gather_index, reduce_index)
    out = _tc_finish(
        x, psums, pcnts, W_v, W_n,
        gamma.reshape(C, 1), beta.reshape(C, 1),
        jnp.reshape(alpha, (1, 1)),
    )
    return out[None]
